# Initial kernel scaffold; baseline (speedup 1.0000x reference)
import jax
import jax.numpy as jnp
from jax.experimental import pallas as pl


def kernel(x, edge_index, batch, W0a, b0a, W0b, b0b, W1a, b1a, W1b, b1b, W2a, b2a, W2b, b2b, g0, be0, g1, be1, g2, be2, Wl1, bl1, Wl2, bl2):
    raise NotImplementedError("write your pallas kernel here")


# speedup vs baseline: 4.9453x; 4.9453x over previous
import functools

import jax
import jax.numpy as jnp
from jax import lax
from jax.experimental import pallas as pl
from jax.experimental.pallas import tpu as pltpu
from jax.experimental.pallas import tpu_sc as plsc

N = 10000
E = 320000
G = 512
HID = 128
NPAD = 10240
GACC = 528
NW = 32
EPT = E // NW
K = 80
NCH = EPT // K
RPT = NPAD // 16
BR = 512
NBLK = NPAD // BR

_mesh = plsc.VectorSubcoreMesh(core_axis_name="c", subcore_axis_name="s")


def _make_aggr(D, apply_relu):
    @functools.partial(
        pl.kernel,
        mesh=_mesh,
        compiler_params=pltpu.CompilerParams(use_tc_tiling_on_sc=False),
        out_type=jax.ShapeDtypeStruct((2, NPAD, D), jnp.float32),
        scratch_types=[
            pltpu.VMEM((2, K), jnp.int32),
            pltpu.VMEM((2, K), jnp.int32),
            pltpu.VMEM((2, K, D), jnp.float32),
            pltpu.VMEM_SHARED((NPAD, D), jnp.float32),
            pltpu.SemaphoreType.DMA,
        ],
    )
    def aggr(h_hbm, src_hbm, dst_hbm, out_hbm, srcb, dstb, rowb, acc, sem):
        c = lax.axis_index("c")
        s = lax.axis_index("s")
        wid = s * 2 + c
        zero = jnp.zeros((16,), jnp.float32)
        for r in range(K):
            for j in range(D // 16):
                rowb[0, r, pl.ds(j * 16, 16)] = zero
        for t in range(RPT // K):
            pltpu.sync_copy(rowb.at[0], acc.at[pl.ds(s * RPT + t * K, K)])
        plsc.subcore_barrier()

        e0 = wid * EPT

        def body(i, carry):
            base = e0 + i * K
            pltpu.sync_copy(src_hbm.at[pl.ds(base, K)], srcb.at[0])
            pltpu.sync_copy(dst_hbm.at[pl.ds(base, K)], dstb.at[0])
            pltpu.async_copy(h_hbm.at[srcb.at[0]], rowb.at[0], sem).wait()
            if apply_relu:
                for r in range(K):
                    for j in range(D // 16):
                        v = rowb[0, r, pl.ds(j * 16, 16)]
                        rowb[0, r, pl.ds(j * 16, 16)] = jnp.maximum(v, 0.0)
            pltpu.sync_copy(rowb.at[0], acc.at[dstb.at[0]], add=True)
            return carry

        lax.fori_loop(0, NCH, body, 0)
        plsc.subcore_barrier()
        r0 = s * RPT
        pltpu.sync_copy(acc.at[pl.ds(r0, RPT)], out_hbm.at[c, pl.ds(r0, RPT)])

    return aggr


_aggr16 = _make_aggr(16, True)
_aggr128 = _make_aggr(128, False)

_KP = 80
_RPP = NPAD // NW
_GRPT = GACC // 16


@functools.partial(
    pl.kernel,
    mesh=_mesh,
    compiler_params=pltpu.CompilerParams(use_tc_tiling_on_sc=False),
    out_type=(
        jax.ShapeDtypeStruct((2, GACC, HID), jnp.float32),
        jax.ShapeDtypeStruct((2, GACC, 16), jnp.float32),
    ),
    scratch_types=[
        pltpu.VMEM((_KP,), jnp.int32),
        pltpu.VMEM((_KP, HID), jnp.float32),
        pltpu.VMEM((_KP, 16), jnp.float32),
        pltpu.VMEM((_GRPT, 16), jnp.float32),
        pltpu.VMEM_SHARED((GACC, HID), jnp.float32),
        pltpu.VMEM_SHARED((GACC, 16), jnp.float32),
    ],
)
def _pool(h_hbm, b_hbm, outs_hbm, outc_hbm, idxb, rowb, oneb, zc, accs, accc):
    c = lax.axis_index("c")
    s = lax.axis_index("s")
    wid = s * 2 + c
    zero = jnp.zeros((16,), jnp.float32)
    one = jnp.ones((16,), jnp.float32)
    for r in range(_KP):
        for j in range(HID // 16):
            rowb[r, pl.ds(j * 16, 16)] = zero
        oneb[r, pl.ds(0, 16)] = one
    for r in range(_GRPT):
        zc[r, pl.ds(0, 16)] = zero
    pltpu.sync_copy(rowb.at[pl.ds(0, _GRPT)], accs.at[pl.ds(s * _GRPT, _GRPT)])
    pltpu.sync_copy(zc, accc.at[pl.ds(s * _GRPT, _GRPT)])
    plsc.subcore_barrier()

    for i in range(_RPP // _KP):
        base = wid * _RPP + i * _KP
        pltpu.sync_copy(h_hbm.at[pl.ds(base, _KP)], rowb)
        pltpu.sync_copy(b_hbm.at[pl.ds(base, _KP)], idxb)
        pltpu.sync_copy(rowb, accs.at[idxb], add=True)
        pltpu.sync_copy(oneb, accc.at[idxb], add=True)
    plsc.subcore_barrier()
    r0 = s * _GRPT
    pltpu.sync_copy(accs.at[pl.ds(r0, _GRPT)], outs_hbm.at[c, pl.ds(r0, _GRPT)])
    pltpu.sync_copy(accc.at[pl.ds(r0, _GRPT)], outc_hbm.at[c, pl.ds(r0, _GRPT)])


def _make_mlp_stats(d_in):
    def body(h_ref, p_ref, wa_ref, ba_ref, wb_ref, bb_ref,
             v_ref, s1_ref, s2_ref):
        i = pl.program_id(0)
        t = h_ref[...] + p_ref[0] + p_ref[1]
        u = jnp.maximum(
            jnp.dot(t, wa_ref[...], preferred_element_type=jnp.float32)
            + ba_ref[...], 0.0)
        v = (jnp.dot(u, wb_ref[...], preferred_element_type=jnp.float32)
             + bb_ref[...])
        v_ref[...] = v
        rows = lax.broadcasted_iota(jnp.int32, (BR, 1), 0) + i * BR
        vm = jnp.where(rows < N, v, 0.0)

        @pl.when(i == 0)
        def _():
            s1_ref[...] = jnp.zeros_like(s1_ref)
            s2_ref[...] = jnp.zeros_like(s2_ref)

        s1_ref[...] += jnp.sum(vm, axis=0, keepdims=True)
        s2_ref[...] += jnp.sum(vm * vm, axis=0, keepdims=True)

    return pl.pallas_call(
        body,
        grid=(NBLK,),
        in_specs=[
            pl.BlockSpec((BR, d_in), lambda i: (i, 0)),
            pl.BlockSpec((2, BR, d_in), lambda i: (0, i, 0)),
            pl.BlockSpec((d_in, HID), lambda i: (0, 0)),
            pl.BlockSpec((1, HID), lambda i: (0, 0)),
            pl.BlockSpec((HID, HID), lambda i: (0, 0)),
            pl.BlockSpec((1, HID), lambda i: (0, 0)),
        ],
        out_specs=[
            pl.BlockSpec((BR, HID), lambda i: (i, 0)),
            pl.BlockSpec((1, HID), lambda i: (0, 0)),
            pl.BlockSpec((1, HID), lambda i: (0, 0)),
        ],
        out_shape=[
            jax.ShapeDtypeStruct((NPAD, HID), jnp.float32),
            jax.ShapeDtypeStruct((1, HID), jnp.float32),
            jax.ShapeDtypeStruct((1, HID), jnp.float32),
        ],
    )


_mlp16 = _make_mlp_stats(16)
_mlp128 = _make_mlp_stats(128)


def _bn_body(v_ref, s1_ref, s2_ref, g_ref, be_ref, h_ref):
    m = s1_ref[...] * (1.0 / N)
    var = s2_ref[...] * (1.0 / N) - m * m
    inv = lax.rsqrt(var + 1e-5)
    h_ref[...] = jnp.maximum(
        (v_ref[...] - m) * inv * g_ref[...] + be_ref[...], 0.0)


_bn = pl.pallas_call(
    _bn_body,
    grid=(NBLK,),
    in_specs=[
        pl.BlockSpec((BR, HID), lambda i: (i, 0)),
        pl.BlockSpec((1, HID), lambda i: (0, 0)),
        pl.BlockSpec((1, HID), lambda i: (0, 0)),
        pl.BlockSpec((1, HID), lambda i: (0, 0)),
        pl.BlockSpec((1, HID), lambda i: (0, 0)),
    ],
    out_specs=pl.BlockSpec((BR, HID), lambda i: (i, 0)),
    out_shape=jax.ShapeDtypeStruct((NPAD, HID), jnp.float32),
)


def _head_body(ps_ref, pc_ref, wl1_ref, bl1_ref, wl2_ref, bl2_ref, y_ref):
    ssum = ps_ref[0, :G, :] + ps_ref[1, :G, :]
    cnt = pc_ref[0, :G, 0:1] + pc_ref[1, :G, 0:1]
    hg = ssum / jnp.maximum(cnt, 1.0)
    u = jnp.maximum(
        jnp.dot(hg, wl1_ref[...], preferred_element_type=jnp.float32)
        + bl1_ref[...], 0.0)
    y_ref[...] = (jnp.dot(u, wl2_ref[...], preferred_element_type=jnp.float32)
                  + bl2_ref[...])


_head = pl.pallas_call(
    _head_body,
    out_shape=jax.ShapeDtypeStruct((G, HID), jnp.float32),
)


def kernel(x, edge_index, batch, W0a, b0a, W0b, b0b, W1a, b1a, W1b, b1b,
           W2a, b2a, W2b, b2b, g0, be0, g1, be1, g2, be2, Wl1, bl1, Wl2, bl2):
    src = edge_index[0]
    dst = edge_index[1]
    x16 = jnp.pad(x, ((0, NPAD - N), (0, 16 - x.shape[1])))
    batchp = jnp.concatenate(
        [batch, jnp.full((NPAD - N,), G, jnp.int32)])
    W0a16 = jnp.pad(W0a, ((0, 16 - W0a.shape[0]), (0, 0)))
    Wl2p = jnp.pad(Wl2, ((0, 0), (0, HID - Wl2.shape[1])))
    bl2p = jnp.pad(bl2, (0, HID - bl2.shape[0]))[None, :]

    p0 = _aggr16(x16, src, dst)
    v, s1, s2 = _mlp16(x16, p0, W0a16, b0a[None, :], W0b, b0b[None, :])
    h = _bn(v, s1, s2, g0[None, :], be0[None, :])

    p1 = _aggr128(h, src, dst)
    v, s1, s2 = _mlp128(h, p1, W1a, b1a[None, :], W1b, b1b[None, :])
    h = _bn(v, s1, s2, g1[None, :], be1[None, :])

    p2 = _aggr128(h, src, dst)
    v, s1, s2 = _mlp128(h, p2, W2a, b2a[None, :], W2b, b2b[None, :])
    h = _bn(v, s1, s2, g2[None, :], be2[None, :])

    ps, pc = _pool(h, batchp)
    y = _head(ps, pc, Wl1, bl1[None, :], Wl2p, bl2p)
    return y[:, :1]


# speedup vs baseline: 6.9446x; 1.4043x over previous
import functools

import jax
import jax.numpy as jnp
from jax import lax
from jax.experimental import pallas as pl
from jax.experimental.pallas import tpu as pltpu
from jax.experimental.pallas import tpu_sc as plsc

N = 10000
E = 320000
G = 512
HID = 128
NPAD = 10240
GACC = 528
NW = 32
EPT = E // NW
K = 40
NCH = EPT // K
RPT = NPAD // 16
BR = 512
NBLK = NPAD // BR

_mesh = plsc.VectorSubcoreMesh(core_axis_name="c", subcore_axis_name="s")


NBUF = 5
NGRP = NCH // NBUF
RPTA = N // 16


def _make_aggr(D, apply_relu):
    @functools.partial(
        pl.kernel,
        mesh=_mesh,
        compiler_params=pltpu.CompilerParams(use_tc_tiling_on_sc=False),
        out_type=jax.ShapeDtypeStruct((2, NPAD, D), jnp.float32),
        scratch_types=[
            pltpu.VMEM((NBUF, K), jnp.int32),
            pltpu.VMEM((NBUF, K), jnp.int32),
            pltpu.VMEM((NBUF, K, D), jnp.float32),
            pltpu.VMEM_SHARED((N, D), jnp.float32),
            pltpu.SemaphoreType.DMA,
            pltpu.SemaphoreType.DMA,
        ],
    )
    def aggr(h_hbm, src_hbm, dst_hbm, out_hbm, srcb, dstb, rowb, acc, gsem, ssem):
        c = lax.axis_index("c")
        s = lax.axis_index("s")
        wid = s * 2 + c

        def zero_buf0():
            zero = jnp.zeros((16,), jnp.float32)
            for r in range(K):
                for j in range(D // 16):
                    rowb[0, r, pl.ds(j * 16, 16)] = zero

        zero_buf0()
        a0 = s * RPTA
        for t in range(RPTA // K):
            pltpu.sync_copy(rowb.at[0], acc.at[pl.ds(a0 + t * K, K)])
        if RPTA % K:
            pltpu.sync_copy(rowb.at[0, pl.ds(0, RPTA % K)],
                            acc.at[pl.ds(a0 + (RPTA // K) * K, RPTA % K)])
        plsc.subcore_barrier()

        row0 = wid * NCH

        def body(g, carry):
            pltpu.sync_copy(src_hbm.at[pl.ds(row0 + g * NBUF, NBUF)], srcb)
            pltpu.sync_copy(dst_hbm.at[pl.ds(row0 + g * NBUF, NBUF)], dstb)
            copies = []
            for b in range(NBUF):
                copies.append(pltpu.async_copy(
                    h_hbm.at[srcb.at[b]], rowb.at[b], gsem))
            for cp in copies:
                cp.wait()
            if apply_relu:
                for b in range(NBUF):
                    for r in range(K):
                        for j in range(D // 16):
                            v = rowb[b, r, pl.ds(j * 16, 16)]
                            rowb[b, r, pl.ds(j * 16, 16)] = jnp.maximum(v, 0.0)
            copies = []
            for b in range(NBUF):
                copies.append(pltpu.async_copy(
                    rowb.at[b], acc.at[dstb.at[b]], ssem, add=True))
            for cp in copies:
                cp.wait()
            return carry

        lax.fori_loop(0, NGRP, body, 0)
        plsc.subcore_barrier()
        r0 = s * RPTA
        pltpu.sync_copy(acc.at[pl.ds(r0, RPTA)], out_hbm.at[c, pl.ds(r0, RPTA)])
        @pl.when(s == 15)
        def _():
            zero_buf0()
            for t in range((NPAD - N) // K):
                pltpu.sync_copy(rowb.at[0],
                                out_hbm.at[c, pl.ds(N + t * K, K)])

    return aggr


_aggr16 = _make_aggr(16, True)
_aggr128 = _make_aggr(128, False)

_KP = 80
_RPP = NPAD // NW
_GRPT = GACC // 16


@functools.partial(
    pl.kernel,
    mesh=_mesh,
    compiler_params=pltpu.CompilerParams(use_tc_tiling_on_sc=False),
    out_type=(
        jax.ShapeDtypeStruct((2, GACC, HID), jnp.float32),
        jax.ShapeDtypeStruct((2, GACC, 16), jnp.float32),
    ),
    scratch_types=[
        pltpu.VMEM((_KP,), jnp.int32),
        pltpu.VMEM((_KP, HID), jnp.float32),
        pltpu.VMEM((_KP, 16), jnp.float32),
        pltpu.VMEM((_GRPT, 16), jnp.float32),
        pltpu.VMEM_SHARED((GACC, HID), jnp.float32),
        pltpu.VMEM_SHARED((GACC, 16), jnp.float32),
    ],
)
def _pool(h_hbm, b_hbm, outs_hbm, outc_hbm, idxb, rowb, oneb, zc, accs, accc):
    c = lax.axis_index("c")
    s = lax.axis_index("s")
    wid = s * 2 + c
    zero = jnp.zeros((16,), jnp.float32)
    one = jnp.ones((16,), jnp.float32)
    for r in range(_KP):
        for j in range(HID // 16):
            rowb[r, pl.ds(j * 16, 16)] = zero
        oneb[r, pl.ds(0, 16)] = one
    for r in range(_GRPT):
        zc[r, pl.ds(0, 16)] = zero
    pltpu.sync_copy(rowb.at[pl.ds(0, _GRPT)], accs.at[pl.ds(s * _GRPT, _GRPT)])
    pltpu.sync_copy(zc, accc.at[pl.ds(s * _GRPT, _GRPT)])
    plsc.subcore_barrier()

    for i in range(_RPP // _KP):
        base = wid * _RPP + i * _KP
        pltpu.sync_copy(h_hbm.at[pl.ds(base, _KP)], rowb)
        pltpu.sync_copy(b_hbm.at[pl.ds(base, _KP)], idxb)
        pltpu.sync_copy(rowb, accs.at[idxb], add=True)
        pltpu.sync_copy(oneb, accc.at[idxb], add=True)
    plsc.subcore_barrier()
    r0 = s * _GRPT
    pltpu.sync_copy(accs.at[pl.ds(r0, _GRPT)], outs_hbm.at[c, pl.ds(r0, _GRPT)])
    pltpu.sync_copy(accc.at[pl.ds(r0, _GRPT)], outc_hbm.at[c, pl.ds(r0, _GRPT)])


_PREC = lax.Precision.DEFAULT


def _make_mlp_stats(d_in):
    def body(h_ref, p_ref, wa_ref, ba_ref, wb_ref, bb_ref, v_ref, s1_ref):
        i = pl.program_id(0)
        t = h_ref[...] + p_ref[0] + p_ref[1]
        u = jnp.maximum(
            jnp.dot(t, wa_ref[...], preferred_element_type=jnp.float32,
                    precision=_PREC) + ba_ref[...], 0.0)
        v = (jnp.dot(u, wb_ref[...], preferred_element_type=jnp.float32,
                     precision=_PREC) + bb_ref[...])
        v_ref[...] = v
        rows = lax.broadcasted_iota(jnp.int32, (BR, 1), 0) + i * BR
        vm = jnp.where(rows < N, v, 0.0)

        @pl.when(i == 0)
        def _():
            s1_ref[...] = jnp.zeros_like(s1_ref)

        s1_ref[...] += jnp.sum(vm, axis=0, keepdims=True)

    return pl.pallas_call(
        body,
        grid=(NBLK,),
        in_specs=[
            pl.BlockSpec((BR, d_in), lambda i: (i, 0)),
            pl.BlockSpec((2, BR, d_in), lambda i: (0, i, 0)),
            pl.BlockSpec((d_in, HID), lambda i: (0, 0)),
            pl.BlockSpec((1, HID), lambda i: (0, 0)),
            pl.BlockSpec((HID, HID), lambda i: (0, 0)),
            pl.BlockSpec((1, HID), lambda i: (0, 0)),
        ],
        out_specs=[
            pl.BlockSpec((BR, HID), lambda i: (i, 0)),
            pl.BlockSpec((1, HID), lambda i: (0, 0)),
        ],
        out_shape=[
            jax.ShapeDtypeStruct((NPAD, HID), jnp.float32),
            jax.ShapeDtypeStruct((1, HID), jnp.float32),
        ],
    )


_mlp16 = _make_mlp_stats(16)
_mlp128 = _make_mlp_stats(128)


def _var_body(v_ref, s1_ref, sq_ref):
    i = pl.program_id(0)
    m = s1_ref[...] / N
    rows = lax.broadcasted_iota(jnp.int32, (BR, 1), 0) + i * BR
    d = jnp.where(rows < N, v_ref[...] - m, 0.0)

    @pl.when(i == 0)
    def _():
        sq_ref[...] = jnp.zeros_like(sq_ref)

    sq_ref[...] += jnp.sum(d * d, axis=0, keepdims=True)


_var = pl.pallas_call(
    _var_body,
    grid=(NBLK,),
    in_specs=[
        pl.BlockSpec((BR, HID), lambda i: (i, 0)),
        pl.BlockSpec((1, HID), lambda i: (0, 0)),
    ],
    out_specs=pl.BlockSpec((1, HID), lambda i: (0, 0)),
    out_shape=jax.ShapeDtypeStruct((1, HID), jnp.float32),
)


def _bn_body(v_ref, s1_ref, sq_ref, g_ref, be_ref, h_ref):
    m = s1_ref[...] / N
    var = sq_ref[...] / N
    h_ref[...] = jnp.maximum(
        (v_ref[...] - m) / jnp.sqrt(var + 1e-5) * g_ref[...] + be_ref[...],
        0.0)


_bn = pl.pallas_call(
    _bn_body,
    grid=(NBLK,),
    in_specs=[
        pl.BlockSpec((BR, HID), lambda i: (i, 0)),
        pl.BlockSpec((1, HID), lambda i: (0, 0)),
        pl.BlockSpec((1, HID), lambda i: (0, 0)),
        pl.BlockSpec((1, HID), lambda i: (0, 0)),
        pl.BlockSpec((1, HID), lambda i: (0, 0)),
    ],
    out_specs=pl.BlockSpec((BR, HID), lambda i: (i, 0)),
    out_shape=jax.ShapeDtypeStruct((NPAD, HID), jnp.float32),
)


def _head_body(ps_ref, pc_ref, wl1_ref, bl1_ref, wl2_ref, bl2_ref, y_ref):
    ssum = ps_ref[0, :G, :] + ps_ref[1, :G, :]
    cnt = pc_ref[0, :G, 0:1] + pc_ref[1, :G, 0:1]
    hg = ssum / jnp.maximum(cnt, 1.0)
    u = jnp.maximum(
        jnp.dot(hg, wl1_ref[...], preferred_element_type=jnp.float32,
                precision=_PREC) + bl1_ref[...], 0.0)
    y_ref[...] = (jnp.dot(u, wl2_ref[...], preferred_element_type=jnp.float32,
                          precision=_PREC) + bl2_ref[...])


_head = pl.pallas_call(
    _head_body,
    out_shape=jax.ShapeDtypeStruct((G, HID), jnp.float32),
)


def kernel(x, edge_index, batch, W0a, b0a, W0b, b0b, W1a, b1a, W1b, b1b,
           W2a, b2a, W2b, b2b, g0, be0, g1, be1, g2, be2, Wl1, bl1, Wl2, bl2):
    src = edge_index[0].reshape(E // K, K)
    dst = edge_index[1].reshape(E // K, K)
    x16 = jnp.pad(x, ((0, NPAD - N), (0, 16 - x.shape[1])))
    batchp = jnp.concatenate(
        [batch, jnp.full((NPAD - N,), G, jnp.int32)])
    W0a16 = jnp.pad(W0a, ((0, 16 - W0a.shape[0]), (0, 0)))
    Wl2p = jnp.pad(Wl2, ((0, 0), (0, HID - Wl2.shape[1])))
    bl2p = jnp.pad(bl2, (0, HID - bl2.shape[0]))[None, :]

    p0 = _aggr16(x16, src, dst)
    v, s1 = _mlp16(x16, p0, W0a16, b0a[None, :], W0b, b0b[None, :])
    h = _bn(v, s1, _var(v, s1), g0[None, :], be0[None, :])

    p1 = _aggr128(h, src, dst)
    v, s1 = _mlp128(h, p1, W1a, b1a[None, :], W1b, b1b[None, :])
    h = _bn(v, s1, _var(v, s1), g1[None, :], be1[None, :])

    p2 = _aggr128(h, src, dst)
    v, s1 = _mlp128(h, p2, W2a, b2a[None, :], W2b, b2b[None, :])
    h = _bn(v, s1, _var(v, s1), g2[None, :], be2[None, :])

    ps, pc = _pool(h, batchp)
    y = _head(ps, pc, Wl1, bl1[None, :], Wl2p, bl2p)
    return y[:, :1]
